# baseline (speedup 1.0000x reference)
import functools

import jax
import jax.numpy as jnp
from jax import lax
from jax.experimental import pallas as pl
from jax.experimental.pallas import tpu as pltpu
from jax.experimental.pallas import tpu_sc as plsc

N, D = 32768, 1024
MBLK = 2048
NB = N // MBLK
NC, NS = 2, 16
NW = NC * NS
RW = N // NW
G = 16
NBUF = 4
NCH = RW // G

def _mask_body(x_ref, flags_ref, tails_ref, carry_ref):
    i = pl.program_id(0)

    @pl.when(i == 0)
    def _():
        carry_ref[0] = 0

    z = x_ref[...] * 0.0
    s = jnp.dot(z, jnp.ones((D, 1), jnp.float32),
                preferred_element_type=jnp.float32)
    f = (s == 0.0).astype(jnp.int32)
    flags_ref[...] = f
    c0 = carry_ref[0]
    subs = [c0 + jnp.sum(f[: (j + 1) * RW]) for j in range(MBLK // RW)]
    tails_ref[...] = jnp.stack(subs).reshape(1, 1, MBLK // RW)
    carry_ref[0] = subs[-1]


_sc_mesh = plsc.VectorSubcoreMesh(core_axis_name="c", subcore_axis_name="s")


@functools.partial(
    pl.kernel,
    mesh=_sc_mesh,
    compiler_params=pltpu.CompilerParams(needs_layout_passes=False),
    out_type=jax.ShapeDtypeStruct((N, D), jnp.float32),
    scratch_types=[
        pltpu.VMEM((NW + 16,), jnp.int32),
        pltpu.VMEM((RW,), jnp.int32),
        pltpu.VMEM((RW,), jnp.int32),
        pltpu.VMEM((NBUF, G, D), jnp.float32),
        [pltpu.SemaphoreType.DMA] * NBUF,
        [pltpu.SemaphoreType.DMA] * NBUF,
        pltpu.SemaphoreType.DMA,
    ],
)
def _sc_filter_gather(x_hbm, flags_hbm, tails_hbm, out_hbm,
                      tails_v, fl_v, idxm_v, rows_v, gsems, wsems, sem2):
    wid = lax.axis_index("s") * NC + lax.axis_index("c")
    mylo = wid * RW
    myhi = mylo + RW
    iota16 = lax.iota(jnp.int32, 16)
    z16 = jnp.zeros((16,), jnp.int32)

    pltpu.async_copy(tails_hbm, tails_v.at[pl.ds(0, NW + 1)], sem2).wait()

    tw = tails_v[pl.ds(wid, 16)]
    ident = jnp.logical_and(tw[0] == mylo, tw[1] == myhi)

    @pl.when(ident)
    def _():
        pltpu.async_copy(
            x_hbm.at[pl.ds(mylo, RW)], out_hbm.at[pl.ds(mylo, RW)],
            sem2).wait()

    @pl.when(jnp.logical_not(ident))
    def _():
        _slow_path(x_hbm, flags_hbm, out_hbm,
                   tails_v, fl_v, idxm_v, rows_v, gsems, wsems, sem2,
                   wid, mylo, myhi, iota16, z16)


def _slow_path(x_hbm, flags_hbm, out_hbm,
               tails_v, fl_v, idxm_v, rows_v, gsems, wsems, sem2,
               wid, mylo, myhi, iota16, z16):
    def zero_body(c, carry):
        idxm_v[pl.ds(pl.multiple_of(c * 16, 16), 16)] = z16
        return carry

    lax.fori_loop(0, RW // 16, zero_body, 0)

    def scan_cond(c):
        k, stop = c
        return jnp.logical_and(k < NW - wid, jnp.logical_not(stop))

    def scan_body(c):
        k, _ = c
        b = wid + k
        base = pl.multiple_of(b * RW, RW)
        pltpu.async_copy(flags_hbm.at[pl.ds(base, RW)], fl_v, sem2).wait()
        t16 = tails_v[pl.ds(b, 16)]

        def inner(c, carry):
            c16 = pl.multiple_of(c * 16, 16)
            f16 = fl_v[pl.ds(c16, 16)]
            p16 = plsc.cumsum(f16) + carry
            t = p16 - 1 - mylo
            m = jnp.logical_and(
                f16 > 0, jnp.logical_and(t >= 0, t < RW))
            vals = iota16 + (base + c16)
            plsc.store_scatter(idxm_v, [t], vals, mask=m)
            return p16[15]

        lax.fori_loop(0, RW // 16, inner, t16[0])
        return k + 1, t16[1] >= myhi

    lax.while_loop(scan_cond, scan_body, (0, False))

    def fire(c, b):
        co = pl.multiple_of(c * G, G)
        pltpu.async_copy(
            x_hbm.at[idxm_v.at[pl.ds(co, G)]], rows_v.at[b], gsems[b])

    fire(0, 0)
    fire(1, 1)

    def gloop(i, carry):
        for j in range(NBUF):
            c = i * NBUF + j
            pltpu.make_async_copy(
                x_hbm.at[pl.ds(0, G)], rows_v.at[j], gsems[j]).wait()
            pltpu.async_copy(
                rows_v.at[j],
                out_hbm.at[pl.ds(pl.multiple_of(mylo + c * G, G), G)],
                wsems[j])
            j2 = (j + 2) % NBUF

            @pl.when(c + 2 < NCH)
            def _():
                @pl.when(c >= 2)
                def _():
                    pltpu.make_async_copy(
                        rows_v.at[j2], out_hbm.at[pl.ds(mylo, G)],
                        wsems[j2]).wait()

                fire(c + 2, j2)

        return carry

    lax.fori_loop(0, NCH // NBUF, gloop, 0)

    for j in range(NBUF):
        pltpu.make_async_copy(
            rows_v.at[j], out_hbm.at[pl.ds(mylo, G)], wsems[j]).wait()


def kernel(x):
    flags2d, tails3d = pl.pallas_call(
        _mask_body,
        grid=(NB,),
        in_specs=[pl.BlockSpec((MBLK, D), lambda i: (i, 0))],
        out_specs=[pl.BlockSpec((MBLK, 1), lambda i: (i, 0)),
                   pl.BlockSpec((1, 1, MBLK // RW), lambda i: (i, 0, 0))],
        out_shape=[jax.ShapeDtypeStruct((N, 1), jnp.int32),
                   jax.ShapeDtypeStruct((NB, 1, MBLK // RW), jnp.int32)],
        scratch_shapes=[pltpu.SMEM((1,), jnp.int32)],
    )(x)
    flags = flags2d.reshape(N)
    tails_ex = jnp.concatenate(
        [jnp.zeros((1,), jnp.int32), tails3d.reshape(N // RW)])
    return _sc_filter_gather(x, flags, tails_ex)


# speedup vs baseline: 24.2881x; 24.2881x over previous
import functools

import jax
import jax.numpy as jnp
from jax import lax
from jax.experimental import pallas as pl
from jax.experimental.pallas import tpu as pltpu
from jax.experimental.pallas import tpu_sc as plsc

N, D = 32768, 1024
MBLK = 2048
NB = N // MBLK
NC, NS = 2, 16
NW = NC * NS
RW = N // NW
G = 16
NBUF = 4
NCH = RW // G

def _mask_body(x_ref, flags_ref, tails_ref, carry_ref):
    i = pl.program_id(0)

    @pl.when(i == 0)
    def _():
        carry_ref[0] = 0

    z = x_ref[...] * 0.0
    s = jnp.dot(z, jnp.ones((D, 1), jnp.float32),
                preferred_element_type=jnp.float32)
    f = (s == 0.0).astype(jnp.int32)
    flags_ref[...] = f
    c0 = carry_ref[0]
    subs = [c0 + jnp.sum(f[: (j + 1) * RW]) for j in range(MBLK // RW)]
    tails_ref[...] = jnp.stack(subs).reshape(1, 1, MBLK // RW)
    carry_ref[0] = subs[-1]


_sc_mesh = plsc.VectorSubcoreMesh(core_axis_name="c", subcore_axis_name="s")


@functools.partial(
    pl.kernel,
    mesh=_sc_mesh,
    compiler_params=pltpu.CompilerParams(needs_layout_passes=False),
    out_type=jax.ShapeDtypeStruct((N, D), jnp.float32),
    scratch_types=[
        pltpu.VMEM((NW + 16,), jnp.int32),
        pltpu.VMEM((RW,), jnp.int32),
        pltpu.VMEM((RW,), jnp.int32),
        pltpu.VMEM((NBUF, G, D), jnp.float32),
        [pltpu.SemaphoreType.DMA] * NBUF,
        [pltpu.SemaphoreType.DMA] * NBUF,
        pltpu.SemaphoreType.DMA,
    ],
)
def _sc_filter_gather(x_hbm, flags_hbm, tails_hbm, out_hbm,
                      tails_v, fl_v, idxm_v, rows_v, gsems, wsems, sem2):
    wid = lax.axis_index("s") * NC + lax.axis_index("c")
    mylo = wid * RW
    myhi = mylo + RW
    iota16 = lax.iota(jnp.int32, 16)
    z16 = jnp.zeros((16,), jnp.int32)

    pltpu.async_copy(tails_hbm, tails_v.at[pl.ds(0, NW + 1)], sem2).wait()

    def zero_body(c, carry):
        idxm_v[pl.ds(pl.multiple_of(c * 16, 16), 16)] = z16
        return carry

    lax.fori_loop(0, RW // 16, zero_body, 0)

    def scan_cond(c):
        k, stop = c
        return jnp.logical_and(k < NW - wid, jnp.logical_not(stop))

    def scan_body(c):
        k, _ = c
        b = wid + k
        base = pl.multiple_of(b * RW, RW)
        pltpu.async_copy(flags_hbm.at[pl.ds(base, RW)], fl_v, sem2).wait()
        t16 = tails_v[pl.ds(b, 16)]

        def inner(c, carry):
            c16 = pl.multiple_of(c * 16, 16)
            f16 = fl_v[pl.ds(c16, 16)]
            p16 = plsc.cumsum(f16) + carry
            t = p16 - 1 - mylo
            m = jnp.logical_and(
                f16 > 0, jnp.logical_and(t >= 0, t < RW))
            vals = iota16 + (base + c16)
            plsc.store_scatter(idxm_v, [t], vals, mask=m)
            return p16[15]

        lax.fori_loop(0, RW // 16, inner, t16[0])
        return k + 1, t16[1] >= myhi

    lax.while_loop(scan_cond, scan_body, (0, False))

    def fire(c, b):
        co = pl.multiple_of(c * G, G)
        pltpu.async_copy(
            x_hbm.at[idxm_v.at[pl.ds(co, G)]], rows_v.at[b], gsems[b])

    fire(0, 0)
    fire(1, 1)

    def gloop(i, carry):
        for j in range(NBUF):
            c = i * NBUF + j
            pltpu.make_async_copy(
                x_hbm.at[pl.ds(0, G)], rows_v.at[j], gsems[j]).wait()
            pltpu.async_copy(
                rows_v.at[j],
                out_hbm.at[pl.ds(pl.multiple_of(mylo + c * G, G), G)],
                wsems[j])
            j2 = (j + 2) % NBUF

            @pl.when(c + 2 < NCH)
            def _():
                @pl.when(c >= 2)
                def _():
                    pltpu.make_async_copy(
                        rows_v.at[j2], out_hbm.at[pl.ds(mylo, G)],
                        wsems[j2]).wait()

                fire(c + 2, j2)

        return carry

    lax.fori_loop(0, NCH // NBUF, gloop, 0)

    for j in range(NBUF):
        pltpu.make_async_copy(
            rows_v.at[j], out_hbm.at[pl.ds(mylo, G)], wsems[j]).wait()


def kernel(x):
    flags2d, tails3d = pl.pallas_call(
        _mask_body,
        grid=(NB,),
        in_specs=[pl.BlockSpec((MBLK, D), lambda i: (i, 0))],
        out_specs=[pl.BlockSpec((MBLK, 1), lambda i: (i, 0)),
                   pl.BlockSpec((1, 1, MBLK // RW), lambda i: (i, 0, 0))],
        out_shape=[jax.ShapeDtypeStruct((N, 1), jnp.int32),
                   jax.ShapeDtypeStruct((NB, 1, MBLK // RW), jnp.int32)],
        scratch_shapes=[pltpu.SMEM((1,), jnp.int32)],
    )(x)
    flags = flags2d.reshape(N)
    tails_ex = jnp.concatenate(
        [jnp.zeros((1,), jnp.int32), tails3d.reshape(N // RW)])
    return _sc_filter_gather(x, flags, tails_ex)
